# baseline (speedup 1.0000x reference)
import functools

import jax
import jax.numpy as jnp
from jax import lax
from jax.experimental import pallas as pl
from jax.experimental.pallas import tpu as pltpu
from jax.experimental.pallas import tpu_sc as plsc

I_SZ = 128
B_SZ = 1024
O_SZ = 64
TAB = 401
XPAD = 416
SKEW = 417
Y_MEAN = 1.05
Y_RANGE = 1.9
SCALE = 10.0
HALF = 512
OG = 32


def _bcast(vec, t):
    return vec.at[jnp.full((16,), t, jnp.int32)].get(mode="promise_in_bounds")


def _sc_run(w_hbm, xt_hbm, z_hbm, out_hbm,
            wrow_v, wy_v, rowidx_v, zrows_v, zskew_v, t2t_v,
            xcol_v, outbuf_v, sem, *, n_workers):
    wid = lax.axis_index("s") * 2 + lax.axis_index("c")
    i_per_w = I_SZ // n_workers
    lane = lax.iota(jnp.int32, 16)

    for k in range(i_per_w):
        i = wid * i_per_w + k
        with jax.named_scope("phase_in_dma"):
            pltpu.sync_copy(w_hbm.at[i], wrow_v)
            pltpu.sync_copy(xt_hbm.at[i], xcol_v)

        for j in range(O_SZ // 16):
            w16 = wrow_v[pl.ds(j * 16, 16)]
            gy = (2.0 * (Y_MEAN - jnp.abs(w16))) / Y_RANGE
            fy = jnp.clip((gy + 1.0) * 0.5 * (TAB - 1), 0.0, float(TAB - 1))
            y0 = fy.astype(jnp.int32)
            wy = fy - y0.astype(jnp.float32)
            y1 = jnp.minimum(y0 + 1, TAB - 1)
            wy_v[pl.ds(j * 16, 16)] = wy
            g, r = divmod(j * 16, OG)
            rowidx_v[pl.ds(g * 2 * OG + r, 16)] = y0
            rowidx_v[pl.ds(g * 2 * OG + OG + r, 16)] = y1

        for g in range(O_SZ // OG):
            with jax.named_scope("phase_zgather"):
                pltpu.async_copy(
                    z_hbm.at[rowidx_v.at[pl.ds(g * 2 * OG, 2 * OG)]],
                    zrows_v, sem).wait()

            with jax.named_scope("phase_skew"):
                @plsc.parallel_loop(0, 2 * OG * (XPAD // 16), unroll=8)
                def skew_copy(n):
                    r = n // (XPAD // 16)
                    c = n % (XPAD // 16)
                    zskew_v[pl.ds(r * SKEW + c * 16, 16)] = \
                        zrows_v[r, pl.ds(c * 16, 16)]

            with jax.named_scope("phase_lut"):
                for j in range(OG // 16):
                    o16 = g * OG + j * 16
                    base0 = (lane + j * 16) * SKEW
                    base1 = base0 + OG * SKEW
                    wyj = wy_v[pl.ds(o16, 16)]

                    @plsc.parallel_loop(0, TAB, unroll=8)
                    def build_x(x):
                        r0 = plsc.load_gather(zskew_v, [base0 + x])
                        r1 = plsc.load_gather(zskew_v, [base1 + x])
                        t2t_v[pl.ds(x * O_SZ + o16, 16)] = \
                            (r0 + (r1 - r0) * wyj) * SCALE

        for h in range(B_SZ // HALF):
            def comp_c(c, carry):
                xv = xcol_v[pl.ds(h * HALF + c * 16, 16)]
                fx = (1.0 - xv) * float(TAB - 1)
                x0 = fx.astype(jnp.int32)
                wx = fx - x0.astype(jnp.float32)
                x1 = jnp.minimum(x0 + 1, TAB - 1)
                x064 = x0 * O_SZ
                x164 = x1 * O_SZ
                for t in range(16):
                    bx0 = _bcast(x064, t)
                    bx1 = _bcast(x164, t)
                    bwx = _bcast(wx, t)
                    for j in range(O_SZ // 16):
                        i0 = bx0 + (lane + j * 16)
                        i1 = bx1 + (lane + j * 16)
                        v0 = plsc.load_gather(t2t_v, [i0])
                        v1 = plsc.load_gather(t2t_v, [i1])
                        outbuf_v[c * 16 + t, pl.ds(j * 16, 16)] = \
                            v0 + (v1 - v0) * bwx
                return carry

            with jax.named_scope("phase_interp"):
                lax.fori_loop(0, HALF // 16, comp_c, 0)
            with jax.named_scope("phase_outdma"):
                pltpu.sync_copy(outbuf_v,
                                out_hbm.at[i, pl.ds(h * HALF, HALF)])


def kernel(weight, x, x_table, z_table):
    del x_table
    xt = x.T
    z_pad = jnp.pad(z_table, ((0, 0), (0, XPAD - TAB)))

    info = plsc.get_sparse_core_info()
    n_workers = info.num_cores * info.num_subcores
    mesh = plsc.VectorSubcoreMesh(core_axis_name="c", subcore_axis_name="s")

    run = functools.partial(
        pl.kernel,
        mesh=mesh,
        compiler_params=pltpu.CompilerParams(
            needs_layout_passes=False, use_tc_tiling_on_sc=False),
        out_type=jax.ShapeDtypeStruct((I_SZ, B_SZ, O_SZ), jnp.float32),
        scratch_types=[
            pltpu.VMEM((O_SZ,), jnp.float32),
            pltpu.VMEM((O_SZ,), jnp.float32),
            pltpu.VMEM((2 * O_SZ,), jnp.int32),
            pltpu.VMEM((2 * OG, XPAD), jnp.float32),
            pltpu.VMEM((2 * OG * SKEW,), jnp.float32),
            pltpu.VMEM((TAB * O_SZ,), jnp.float32),
            pltpu.VMEM((B_SZ,), jnp.float32),
            pltpu.VMEM((HALF, O_SZ), jnp.float32),
            pltpu.SemaphoreType.DMA,
        ],
    )(functools.partial(_sc_run, n_workers=n_workers))
    return run(weight, xt, z_pad)


# speedup vs baseline: 1.4787x; 1.4787x over previous
import functools

import jax
import jax.numpy as jnp
from jax import lax
from jax.experimental import pallas as pl
from jax.experimental.pallas import tpu as pltpu
from jax.experimental.pallas import tpu_sc as plsc

I_SZ = 128
B_SZ = 1024
O_SZ = 64
TAB = 401
XPAD = 416
SKEW = 417
Y_MEAN = 1.05
Y_RANGE = 1.9
SCALE = 10.0
HALF = 512
OG = 32


def _bcast(vec, t):
    return vec.at[jnp.full((16,), t, jnp.int32)].get(mode="promise_in_bounds")


def _sc_run(w_hbm, xt_hbm, z_hbm, out_hbm,
            wrow_v, wy_v, rowidx_v, zrows_v, zskew_v, t2t_v,
            xcol_v, outbuf_v, sem, *, n_workers):
    wid = lax.axis_index("s") * 2 + lax.axis_index("c")
    i_per_w = I_SZ // n_workers
    lane = lax.iota(jnp.int32, 16)

    for k in range(i_per_w):
        i = wid * i_per_w + k
        with jax.named_scope("phase_in_dma"):
            pltpu.sync_copy(w_hbm.at[i], wrow_v)
            pltpu.sync_copy(xt_hbm.at[i], xcol_v)

        for j in range(O_SZ // 16):
            w16 = wrow_v[pl.ds(j * 16, 16)]
            gy = (2.0 * (Y_MEAN - jnp.abs(w16))) / Y_RANGE
            fy = jnp.clip((gy + 1.0) * 0.5 * (TAB - 1), 0.0, float(TAB - 1))
            y0 = fy.astype(jnp.int32)
            wy = fy - y0.astype(jnp.float32)
            y1 = jnp.minimum(y0 + 1, TAB - 1)
            wy_v[pl.ds(j * 16, 16)] = wy
            g, r = divmod(j * 16, OG)
            rowidx_v[pl.ds(g * 2 * OG + r, 16)] = y0
            rowidx_v[pl.ds(g * 2 * OG + OG + r, 16)] = y1

        for g in range(O_SZ // OG):
            with jax.named_scope("phase_zgather"):
                pltpu.async_copy(
                    z_hbm.at[rowidx_v.at[pl.ds(g * 2 * OG, 2 * OG)]],
                    zrows_v, sem).wait()

            with jax.named_scope("phase_skew"):
                @plsc.parallel_loop(0, 2 * OG * (XPAD // 16), unroll=8)
                def skew_copy(n):
                    r = n // (XPAD // 16)
                    c = n % (XPAD // 16)
                    zskew_v[pl.ds(r * SKEW + c * 16, 16)] = \
                        zrows_v[r, pl.ds(c * 16, 16)]

            with jax.named_scope("phase_lut"):
                for j in range(OG // 16):
                    o16 = g * OG + j * 16
                    base0 = (lane + j * 16) * SKEW
                    base1 = base0 + OG * SKEW
                    wyj = wy_v[pl.ds(o16, 16)]

                    @plsc.parallel_loop(0, TAB, unroll=8)
                    def build_x(x):
                        r0 = plsc.load_gather(zskew_v, [base0 + x])
                        r1 = plsc.load_gather(zskew_v, [base1 + x])
                        t2t_v[pl.ds(x * O_SZ + o16, 16)] = \
                            (r0 + (r1 - r0) * wyj) * SCALE

        for h in range(B_SZ // HALF):
            def comp_c(c, carry):
                xv = xcol_v[pl.ds(h * HALF + c * 16, 16)]
                fx = (1.0 - xv) * float(TAB - 1)
                x0 = fx.astype(jnp.int32)
                wx = fx - x0.astype(jnp.float32)
                x1 = jnp.minimum(x0 + 1, TAB - 1)
                x064 = x0 * O_SZ
                x164 = x1 * O_SZ

                @plsc.parallel_loop(0, 16, unroll=4)
                def comp_t(t):
                    bx0 = _bcast(x064, t)
                    bx1 = _bcast(x164, t)
                    bwx = _bcast(wx, t)
                    for j in range(O_SZ // 16):
                        i0 = bx0 + (lane + j * 16)
                        i1 = bx1 + (lane + j * 16)
                        v0 = plsc.load_gather(t2t_v, [i0])
                        v1 = plsc.load_gather(t2t_v, [i1])
                        outbuf_v[c * 16 + t, pl.ds(j * 16, 16)] = \
                            v0 + (v1 - v0) * bwx

                return carry

            with jax.named_scope("phase_interp"):
                lax.fori_loop(0, HALF // 16, comp_c, 0)
            with jax.named_scope("phase_outdma"):
                pltpu.sync_copy(outbuf_v,
                                out_hbm.at[i, pl.ds(h * HALF, HALF)])


def kernel(weight, x, x_table, z_table):
    del x_table
    xt = x.T
    z_pad = jnp.pad(z_table, ((0, 0), (0, XPAD - TAB)))

    info = plsc.get_sparse_core_info()
    n_workers = info.num_cores * info.num_subcores
    mesh = plsc.VectorSubcoreMesh(core_axis_name="c", subcore_axis_name="s")

    run = functools.partial(
        pl.kernel,
        mesh=mesh,
        compiler_params=pltpu.CompilerParams(
            needs_layout_passes=False, use_tc_tiling_on_sc=False),
        out_type=jax.ShapeDtypeStruct((I_SZ, B_SZ, O_SZ), jnp.float32),
        scratch_types=[
            pltpu.VMEM((O_SZ,), jnp.float32),
            pltpu.VMEM((O_SZ,), jnp.float32),
            pltpu.VMEM((2 * O_SZ,), jnp.int32),
            pltpu.VMEM((2 * OG, XPAD), jnp.float32),
            pltpu.VMEM((2 * OG * SKEW,), jnp.float32),
            pltpu.VMEM((TAB * O_SZ,), jnp.float32),
            pltpu.VMEM((B_SZ,), jnp.float32),
            pltpu.VMEM((HALF, O_SZ), jnp.float32),
            pltpu.SemaphoreType.DMA,
        ],
    )(functools.partial(_sc_run, n_workers=n_workers))
    return run(weight, xt, z_pad)


# speedup vs baseline: 1.4831x; 1.0030x over previous
import functools

import jax
import jax.numpy as jnp
from jax import lax
from jax.experimental import pallas as pl
from jax.experimental.pallas import tpu as pltpu
from jax.experimental.pallas import tpu_sc as plsc

I_SZ = 128
B_SZ = 1024
O_SZ = 64
TAB = 401
XPAD = 416
SKEW = 417
Y_MEAN = 1.05
Y_RANGE = 1.9
SCALE = 10.0
HALF = 512
OG = 32


def _bcast(vec, t):
    return vec.at[jnp.full((16,), t, jnp.int32)].get(mode="promise_in_bounds")


def _sc_run(w_hbm, xt_hbm, z_hbm, out_hbm,
            wrow_v, wy_v, rowidx_v, zrows_v, zskew_v, t2t_v,
            xcol_v, outbuf_v, sem, *, n_workers):
    wid = lax.axis_index("s") * 2 + lax.axis_index("c")
    i_per_w = I_SZ // n_workers
    lane = lax.iota(jnp.int32, 16)

    for k in range(i_per_w):
        i = wid * i_per_w + k
        with jax.named_scope("phase_in_dma"):
            pltpu.sync_copy(w_hbm.at[i], wrow_v)
            pltpu.sync_copy(xt_hbm.at[i], xcol_v)

        for j in range(O_SZ // 16):
            w16 = wrow_v[pl.ds(j * 16, 16)]
            gy = (2.0 * (Y_MEAN - jnp.abs(w16))) / Y_RANGE
            fy = jnp.clip((gy + 1.0) * 0.5 * (TAB - 1), 0.0, float(TAB - 1))
            y0 = fy.astype(jnp.int32)
            wy = fy - y0.astype(jnp.float32)
            y1 = jnp.minimum(y0 + 1, TAB - 1)
            wy_v[pl.ds(j * 16, 16)] = wy
            g, r = divmod(j * 16, OG)
            rowidx_v[pl.ds(g * 2 * OG + r, 16)] = y0
            rowidx_v[pl.ds(g * 2 * OG + OG + r, 16)] = y1

        for g in range(O_SZ // OG):
            with jax.named_scope("phase_zgather"):
                pltpu.async_copy(
                    z_hbm.at[rowidx_v.at[pl.ds(g * 2 * OG, 2 * OG)]],
                    zrows_v, sem).wait()

            with jax.named_scope("phase_skew"):
                @plsc.parallel_loop(0, 2 * OG, unroll=2)
                def skew_copy(r):
                    base = r * SKEW
                    for cc in range(XPAD // 16):
                        zskew_v[pl.ds(base + cc * 16, 16)] = \
                            zrows_v[r, pl.ds(cc * 16, 16)]

            with jax.named_scope("phase_lut"):
                for j in range(OG // 16):
                    o16 = g * OG + j * 16
                    base0 = (lane + j * 16) * SKEW
                    base1 = base0 + OG * SKEW
                    wyj = wy_v[pl.ds(o16, 16)]

                    @plsc.parallel_loop(0, TAB, unroll=8)
                    def build_x(x):
                        r0 = plsc.load_gather(zskew_v, [base0 + x])
                        r1 = plsc.load_gather(zskew_v, [base1 + x])
                        t2t_v[pl.ds(x * O_SZ + o16, 16)] = \
                            (r0 + (r1 - r0) * wyj) * SCALE

        for h in range(B_SZ // HALF):
            def comp_c(c, carry):
                xv = xcol_v[pl.ds(h * HALF + c * 16, 16)]
                fx = (1.0 - xv) * float(TAB - 1)
                x0 = fx.astype(jnp.int32)
                wx = fx - x0.astype(jnp.float32)
                x1 = jnp.minimum(x0 + 1, TAB - 1)
                x064 = x0 * O_SZ
                x164 = x1 * O_SZ

                @plsc.parallel_loop(0, 16, unroll=4)
                def comp_t(t):
                    bx0 = _bcast(x064, t)
                    bx1 = _bcast(x164, t)
                    bwx = _bcast(wx, t)
                    for j in range(O_SZ // 16):
                        i0 = bx0 + (lane + j * 16)
                        i1 = bx1 + (lane + j * 16)
                        v0 = plsc.load_gather(t2t_v, [i0])
                        v1 = plsc.load_gather(t2t_v, [i1])
                        outbuf_v[c * 16 + t, pl.ds(j * 16, 16)] = \
                            v0 + (v1 - v0) * bwx

                return carry

            with jax.named_scope("phase_interp"):
                lax.fori_loop(0, HALF // 16, comp_c, 0)
            with jax.named_scope("phase_outdma"):
                pltpu.sync_copy(outbuf_v,
                                out_hbm.at[i, pl.ds(h * HALF, HALF)])


def kernel(weight, x, x_table, z_table):
    del x_table
    xt = x.T
    z_pad = jnp.pad(z_table, ((0, 0), (0, XPAD - TAB)))

    info = plsc.get_sparse_core_info()
    n_workers = info.num_cores * info.num_subcores
    mesh = plsc.VectorSubcoreMesh(core_axis_name="c", subcore_axis_name="s")

    run = functools.partial(
        pl.kernel,
        mesh=mesh,
        compiler_params=pltpu.CompilerParams(
            needs_layout_passes=False, use_tc_tiling_on_sc=False),
        out_type=jax.ShapeDtypeStruct((I_SZ, B_SZ, O_SZ), jnp.float32),
        scratch_types=[
            pltpu.VMEM((O_SZ,), jnp.float32),
            pltpu.VMEM((O_SZ,), jnp.float32),
            pltpu.VMEM((2 * O_SZ,), jnp.int32),
            pltpu.VMEM((2 * OG, XPAD), jnp.float32),
            pltpu.VMEM((2 * OG * SKEW,), jnp.float32),
            pltpu.VMEM((TAB * O_SZ,), jnp.float32),
            pltpu.VMEM((B_SZ,), jnp.float32),
            pltpu.VMEM((HALF, O_SZ), jnp.float32),
            pltpu.SemaphoreType.DMA,
        ],
    )(functools.partial(_sc_run, n_workers=n_workers))
    return run(weight, xt, z_pad)


# speedup vs baseline: 1.5574x; 1.0501x over previous
import functools

import jax
import jax.numpy as jnp
from jax import lax
from jax.experimental import pallas as pl
from jax.experimental.pallas import tpu as pltpu
from jax.experimental.pallas import tpu_sc as plsc

I_SZ = 128
B_SZ = 1024
O_SZ = 64
TAB = 401
XPAD = 416
SKEW = 417
Y_MEAN = 1.05
Y_RANGE = 1.9
SCALE = 10.0
QTR = 512
OG = 16


def _bcast(vec, t):
    return vec.at[jnp.full((16,), t, jnp.int32)].get(mode="promise_in_bounds")


def _sc_run(w_hbm, xt_hbm, z_hbm, out_hbm,
            wrow_v, wy_v, rowidx_v, zrows_v, zskew_v, t2t_v,
            xcol_v, outbuf_v, sem_in, sem_z, sem_o0, sem_o1, *, n_workers):
    wid = lax.axis_index("s") * 2 + lax.axis_index("c")
    i_per_w = I_SZ // n_workers
    i0 = wid * i_per_w
    lane = lax.iota(jnp.int32, 16)
    out_sems = (sem_o0, sem_o1)

    def fy_pass(par):
        for j in range(O_SZ // 16):
            w16 = wrow_v[par, pl.ds(j * 16, 16)]
            gy = (2.0 * (Y_MEAN - jnp.abs(w16))) / Y_RANGE
            fy = jnp.clip((gy + 1.0) * 0.5 * (TAB - 1), 0.0, float(TAB - 1))
            y0 = fy.astype(jnp.int32)
            wy = fy - y0.astype(jnp.float32)
            y1 = jnp.minimum(y0 + 1, TAB - 1)
            wy_v[par, pl.ds(j * 16, 16)] = wy
            rowidx_v[par, pl.ds(j * 2 * OG, 16)] = y0
            rowidx_v[par, pl.ds(j * 2 * OG + OG, 16)] = y1

    def z_gather(par, g):
        return pltpu.async_copy(
            z_hbm.at[rowidx_v.at[par, pl.ds(g * 2 * OG, 2 * OG)]],
            zrows_v, sem_z)

    d_w = pltpu.async_copy(w_hbm.at[i0], wrow_v.at[0], sem_in)
    d_x = pltpu.async_copy(xt_hbm.at[i0], xcol_v.at[0], sem_in)
    d_w.wait()
    d_x.wait()
    fy_pass(0)
    zg = z_gather(0, 0)
    out_pending = [None, None]

    for k in range(i_per_w):
        par, nxt = k % 2, (k + 1) % 2
        i = i0 + k
        if k + 1 < i_per_w:
            d_w = pltpu.async_copy(w_hbm.at[i + 1], wrow_v.at[nxt], sem_in)
            d_x = pltpu.async_copy(xt_hbm.at[i + 1], xcol_v.at[nxt], sem_in)

        for g in range(O_SZ // OG):
            zg.wait()

            @plsc.parallel_loop(0, 2 * OG, unroll=2)
            def skew_copy(r):
                base = r * SKEW
                for cc in range(XPAD // 16):
                    zskew_v[pl.ds(base + cc * 16, 16)] = \
                        zrows_v[r, pl.ds(cc * 16, 16)]

            if g + 1 < O_SZ // OG:
                zg = z_gather(par, g + 1)
            elif k + 1 < i_per_w:
                d_w.wait()
                d_x.wait()
                fy_pass(nxt)
                zg = z_gather(nxt, 0)

            o16 = g * OG
            base0 = lane * SKEW
            base1 = base0 + OG * SKEW
            wyj = wy_v[par, pl.ds(o16, 16)]

            @plsc.parallel_loop(0, TAB, unroll=8)
            def build_x(x):
                r0 = plsc.load_gather(zskew_v, [base0 + x])
                r1 = plsc.load_gather(zskew_v, [base1 + x])
                t2t_v[pl.ds(x * O_SZ + o16, 16)] = \
                    (r0 + (r1 - r0) * wyj) * SCALE

        for q in range(B_SZ // QTR):
            ob = q % 2
            if out_pending[ob] is not None:
                out_pending[ob].wait()

            def comp_c(c, carry):
                xv = xcol_v[par, pl.ds(q * QTR + c * 16, 16)]
                fx = (1.0 - xv) * float(TAB - 1)
                x0 = fx.astype(jnp.int32)
                wx = fx - x0.astype(jnp.float32)
                x1 = jnp.minimum(x0 + 1, TAB - 1)
                x064 = x0 * O_SZ
                x164 = x1 * O_SZ

                @plsc.parallel_loop(0, 16, unroll=4)
                def comp_t(t):
                    bx0 = _bcast(x064, t)
                    bx1 = _bcast(x164, t)
                    bwx = _bcast(wx, t)
                    for j in range(O_SZ // 16):
                        i0_ = bx0 + (lane + j * 16)
                        i1_ = bx1 + (lane + j * 16)
                        v0 = plsc.load_gather(t2t_v, [i0_])
                        v1 = plsc.load_gather(t2t_v, [i1_])
                        outbuf_v[ob, c * 16 + t, pl.ds(j * 16, 16)] = \
                            v0 + (v1 - v0) * bwx

                return carry

            lax.fori_loop(0, QTR // 16, comp_c, 0)
            out_pending[ob] = pltpu.async_copy(
                outbuf_v.at[ob], out_hbm.at[i, pl.ds(q * QTR, QTR)],
                out_sems[ob])

    out_pending[0].wait()
    out_pending[1].wait()


def kernel(weight, x, x_table, z_table):
    del x_table
    xt = x.T
    z_pad = jnp.pad(z_table, ((0, 0), (0, XPAD - TAB)))

    info = plsc.get_sparse_core_info()
    n_workers = info.num_cores * info.num_subcores
    mesh = plsc.VectorSubcoreMesh(core_axis_name="c", subcore_axis_name="s")

    run = functools.partial(
        pl.kernel,
        mesh=mesh,
        compiler_params=pltpu.CompilerParams(
            needs_layout_passes=False, use_tc_tiling_on_sc=False),
        out_type=jax.ShapeDtypeStruct((I_SZ, B_SZ, O_SZ), jnp.float32),
        scratch_types=[
            pltpu.VMEM((2, O_SZ), jnp.float32),
            pltpu.VMEM((2, O_SZ), jnp.float32),
            pltpu.VMEM((2, 2 * O_SZ), jnp.int32),
            pltpu.VMEM((2 * OG, XPAD), jnp.float32),
            pltpu.VMEM((2 * OG * SKEW,), jnp.float32),
            pltpu.VMEM((TAB * O_SZ,), jnp.float32),
            pltpu.VMEM((2, B_SZ), jnp.float32),
            pltpu.VMEM((2, QTR, O_SZ), jnp.float32),
            pltpu.SemaphoreType.DMA,
            pltpu.SemaphoreType.DMA,
            pltpu.SemaphoreType.DMA,
            pltpu.SemaphoreType.DMA,
        ],
    )(functools.partial(_sc_run, n_workers=n_workers))
    return run(weight, xt, z_pad)
